# baseline (speedup 1.0000x reference)
import functools

import jax
import jax.numpy as jnp
from jax import lax
from jax.experimental import pallas as pl
from jax.experimental.pallas import tpu as pltpu
from jax.experimental.pallas import tpu_sc as plsc

N = 10000
NP = 10240
E = 320000
C = 128
NC = 2
NS = 16
NW = NC * NS
EPW = E // NW
BK = 80
NB = 125
CH = 25
NQ = NB // CH
EPAD = NB * BK - EPW
RPS = NP // NS


def _sc_mesh():
    return plsc.VectorSubcoreMesh(core_axis_name="c", subcore_axis_name="s",
                                  num_cores=NC, num_subcores=NS)


@functools.cache
def _build_deg_kernel():
    @functools.partial(
        pl.kernel,
        out_type=jax.ShapeDtypeStruct((NC, NP, C), jnp.float32),
        mesh=_sc_mesh(),
        scratch_types=[
            pltpu.VMEM((NB, BK), jnp.int32),
            pltpu.VMEM((BK, C), jnp.float32),
            pltpu.VMEM_SHARED((NP, C), jnp.float32),
        ],
    )
    def _deg_kernel(dst_hbm, ones_hbm, zeros_hbm, out_hbm, didx, ones_v, acc_sh):
        c = lax.axis_index("c")
        s = lax.axis_index("s")
        wid = c * NS + s
        pltpu.sync_copy(zeros_hbm.at[pl.ds(s * RPS, RPS)],
                        acc_sh.at[pl.ds(s * RPS, RPS)])
        pltpu.sync_copy(ones_hbm, ones_v)
        pltpu.sync_copy(dst_hbm.at[wid], didx)
        plsc.subcore_barrier()

        def body(j, carry):
            pltpu.sync_copy(ones_v, acc_sh.at[didx.at[j]], add=True)
            return carry

        lax.fori_loop(0, NB, body, 0)
        plsc.subcore_barrier()
        pltpu.sync_copy(acc_sh.at[pl.ds(s * RPS, RPS)],
                        out_hbm.at[c, pl.ds(s * RPS, RPS)])

    return _deg_kernel


@functools.cache
def _build_scatter_kernel():
    @functools.partial(
        pl.kernel,
        out_type=jax.ShapeDtypeStruct((NC, NP, C), jnp.float32),
        mesh=_sc_mesh(),
        scratch_types=[
            pltpu.VMEM((2, 2, CH, BK), jnp.int32),
            pltpu.VMEM((BK, C), jnp.float32),
            pltpu.VMEM((BK, C), jnp.float32),
            pltpu.VMEM_SHARED((NP, C), jnp.float32),
            pltpu.SemaphoreType.DMA,
            pltpu.SemaphoreType.DMA,
        ],
    )
    def _scatter_kernel(g_hbm, idx_hbm, zeros_hbm, out_hbm,
                        ibuf, rows_a, rows_b, acc_sh, sem_a, sem_b):
        c = lax.axis_index("c")
        s = lax.axis_index("s")
        wid = c * NS + s
        pltpu.sync_copy(zeros_hbm.at[pl.ds(s * RPS, RPS)],
                        acc_sh.at[pl.ds(s * RPS, RPS)])
        pltpu.sync_copy(idx_hbm.at[wid, 0], ibuf.at[0])
        plsc.subcore_barrier()

        pltpu.async_copy(g_hbm.at[ibuf.at[0, 0, 0]], rows_a, sem_a)

        def body(j, carry):
            q = j // CH
            t = j % CH
            jn = j + 1
            qn = jn // CH
            tn = jn % CH

            @pl.when((t == 0) & (q + 1 < NQ))
            def _():
                pltpu.sync_copy(idx_hbm.at[wid, q + 1],
                                ibuf.at[(q + 1) % 2])

            even = (j % 2) == 0

            @pl.when(even)
            def _():
                pltpu.make_async_copy(
                    g_hbm.at[ibuf.at[q % 2, 0, t]], rows_a, sem_a).wait()

                @pl.when(jn < NB)
                def _():
                    pltpu.async_copy(
                        g_hbm.at[ibuf.at[qn % 2, 0, tn]], rows_b, sem_b)

                pltpu.sync_copy(rows_a, acc_sh.at[ibuf.at[q % 2, 1, t]],
                                add=True)

            @pl.when(jnp.logical_not(even))
            def _():
                pltpu.make_async_copy(
                    g_hbm.at[ibuf.at[q % 2, 0, t]], rows_b, sem_b).wait()

                @pl.when(jn < NB)
                def _():
                    pltpu.async_copy(
                        g_hbm.at[ibuf.at[qn % 2, 0, tn]], rows_a, sem_a)

                pltpu.sync_copy(rows_b, acc_sh.at[ibuf.at[q % 2, 1, t]],
                                add=True)

            return carry

        lax.fori_loop(0, NB, body, 0)
        plsc.subcore_barrier()
        pltpu.sync_copy(acc_sh.at[pl.ds(s * RPS, RPS)],
                        out_hbm.at[c, pl.ds(s * RPS, RPS)])

    return _scatter_kernel


def _dinv(d_ref):
    deg = d_ref[:, 0:1] + d_ref[:, 1:2] + 1.0
    return lax.rsqrt(deg)


def _tc1_body(x_ref, w1_ref, d_ref, g1_ref):
    h = jnp.dot(x_ref[...], w1_ref[...], preferred_element_type=jnp.float32)
    g1_ref[...] = h * _dinv(d_ref)


def _tc2_body(sp_ref, g1_ref, d_ref, b1_ref, w2_ref, g2_ref):
    dinv = _dinv(d_ref)
    ssum = sp_ref[0, :N, :] + sp_ref[1, :N, :]
    x2 = dinv * (ssum + g1_ref[...]) + b1_ref[...]
    x2 = jnp.maximum(x2, 0.0)
    h2 = jnp.dot(x2, w2_ref[...], preferred_element_type=jnp.float32)
    g2_ref[...] = h2 * dinv


def _tc3_body(sp_ref, g2_ref, d_ref, b2_ref, out_ref):
    dinv = _dinv(d_ref)
    ssum = sp_ref[0, :N, :] + sp_ref[1, :N, :]
    out_ref[...] = dinv * (ssum + g2_ref[...]) + b2_ref[...]


_f32 = jnp.float32
_tc1 = pl.pallas_call(_tc1_body, out_shape=jax.ShapeDtypeStruct((N, C), _f32))
_tc2 = pl.pallas_call(_tc2_body, out_shape=jax.ShapeDtypeStruct((N, C), _f32))
_tc3 = pl.pallas_call(_tc3_body, out_shape=jax.ShapeDtypeStruct((N, C), _f32))


def kernel(x, edge_index, W1, b1, W2, b2):
    ei = edge_index.astype(jnp.int32)
    src2 = ei[0].reshape(NW, EPW)
    dst2 = ei[1].reshape(NW, EPW)
    srcp = jnp.concatenate(
        [src2, jnp.zeros((NW, EPAD), jnp.int32)], axis=1).reshape(NW, NB, BK)
    dstp = jnp.concatenate(
        [dst2, jnp.full((NW, EPAD), NP - 1, jnp.int32)], axis=1).reshape(NW, NB, BK)
    idx4 = jnp.stack([srcp.reshape(NW, NQ, CH, BK),
                      dstp.reshape(NW, NQ, CH, BK)], axis=2)
    onesC = jnp.ones((BK, C), _f32)
    zerosC = jnp.zeros((NP, C), _f32)
    b1r = b1.reshape(1, C)
    b2r = b2.reshape(1, C)

    deg_kernel = _build_deg_kernel()
    scatter_kernel = _build_scatter_kernel()

    degp = deg_kernel(dstp, onesC, zerosC)
    d2 = jnp.stack([degp[0, :N, 0], degp[1, :N, 0]], axis=1)

    g1 = _tc1(x, W1, d2)
    s1 = scatter_kernel(g1, idx4, zerosC)
    g2 = _tc2(s1, g1, d2, b1r, W2)
    s2 = scatter_kernel(g2, idx4, zerosC)
    out = _tc3(s2, g2, d2, b2r)
    return out


# speedup vs baseline: 1.3165x; 1.3165x over previous
import functools

import jax
import jax.numpy as jnp
from jax import lax
from jax.experimental import pallas as pl
from jax.experimental.pallas import tpu as pltpu
from jax.experimental.pallas import tpu_sc as plsc

N = 10000
NP = 10240
E = 320000
C = 128
NC = 2
NS = 16
NW = NC * NS
EPW = E // NW
BK = 80
NB = 125
CH = 25
NQ = NB // CH
EPAD = NB * BK - EPW
RPS = NP // NS


def _sc_mesh():
    return plsc.VectorSubcoreMesh(core_axis_name="c", subcore_axis_name="s",
                                  num_cores=NC, num_subcores=NS)


@functools.cache
def _build_deg_kernel():
    @functools.partial(
        pl.kernel,
        out_type=jax.ShapeDtypeStruct((NC, NP, C), jnp.float32),
        mesh=_sc_mesh(),
        scratch_types=[
            pltpu.VMEM((NB, BK), jnp.int32),
            pltpu.VMEM((BK, C), jnp.float32),
            pltpu.VMEM_SHARED((NP, C), jnp.float32),
        ],
    )
    def _deg_kernel(dst_hbm, ones_hbm, zeros_hbm, out_hbm, didx, ones_v, acc_sh):
        c = lax.axis_index("c")
        s = lax.axis_index("s")
        wid = c * NS + s
        pltpu.sync_copy(zeros_hbm.at[pl.ds(s * RPS, RPS)],
                        acc_sh.at[pl.ds(s * RPS, RPS)])
        pltpu.sync_copy(ones_hbm, ones_v)
        pltpu.sync_copy(dst_hbm.at[wid], didx)
        plsc.subcore_barrier()

        def body(j, carry):
            pltpu.sync_copy(ones_v, acc_sh.at[didx.at[j]], add=True)
            return carry

        lax.fori_loop(0, NB, body, 0)
        plsc.subcore_barrier()
        pltpu.sync_copy(acc_sh.at[pl.ds(s * RPS, RPS)],
                        out_hbm.at[c, pl.ds(s * RPS, RPS)])

    return _deg_kernel


@functools.cache
def _build_scatter_kernel():
    @functools.partial(
        pl.kernel,
        out_type=jax.ShapeDtypeStruct((NC, NP, C), jnp.float32),
        mesh=_sc_mesh(),
        scratch_types=[
            pltpu.VMEM((2, 2, CH, BK), jnp.int32),
            pltpu.VMEM((BK, C), jnp.float32),
            pltpu.VMEM((BK, C), jnp.float32),
            pltpu.VMEM((BK, C), jnp.float32),
            pltpu.VMEM_SHARED((NP, C), jnp.float32),
            pltpu.SemaphoreType.DMA,
            pltpu.SemaphoreType.DMA,
            pltpu.SemaphoreType.DMA,
        ],
    )
    def _scatter_kernel(g_hbm, idx_hbm, zeros_hbm, out_hbm,
                        ibuf, rows0, rows1, rows2, acc_sh, sem0, sem1, sem2):
        c = lax.axis_index("c")
        s = lax.axis_index("s")
        wid = c * NS + s
        pltpu.sync_copy(zeros_hbm.at[pl.ds(s * RPS, RPS)],
                        acc_sh.at[pl.ds(s * RPS, RPS)])
        pltpu.sync_copy(idx_hbm.at[wid, 0], ibuf.at[0])
        plsc.subcore_barrier()

        rows = (rows0, rows1, rows2)
        sems = (sem0, sem1, sem2)

        pltpu.async_copy(g_hbm.at[ibuf.at[0, 0, 0]], rows0, sem0)
        pltpu.async_copy(g_hbm.at[ibuf.at[0, 0, 1]], rows1, sem1)

        def body(j, carry):
            q = j // CH
            t = j % CH

            @pl.when((t == 0) & (q + 1 < NQ))
            def _():
                pltpu.sync_copy(idx_hbm.at[wid, q + 1],
                                ibuf.at[(q + 1) % 2])

            j2 = j + 2
            q2 = j2 // CH
            t2 = j2 % CH

            for k in range(3):
                @pl.when(j % 3 == k)
                def _(k=k):
                    pltpu.make_async_copy(
                        g_hbm.at[ibuf.at[q % 2, 0, t]], rows[k], sems[k]).wait()

                    @pl.when(j2 < NB)
                    def _():
                        pltpu.async_copy(
                            g_hbm.at[ibuf.at[q2 % 2, 0, t2]],
                            rows[(k + 2) % 3], sems[(k + 2) % 3])

                    pltpu.sync_copy(rows[k], acc_sh.at[ibuf.at[q % 2, 1, t]],
                                    add=True)

            return carry

        lax.fori_loop(0, NB, body, 0)
        plsc.subcore_barrier()
        pltpu.sync_copy(acc_sh.at[pl.ds(s * RPS, RPS)],
                        out_hbm.at[c, pl.ds(s * RPS, RPS)])

    return _scatter_kernel


def _dinv(d_ref):
    deg = d_ref[:, 0:1] + d_ref[:, 1:2] + 1.0
    return lax.rsqrt(deg)


def _tc1_body(x_ref, w1_ref, d_ref, g1_ref):
    h = jnp.dot(x_ref[...], w1_ref[...], preferred_element_type=jnp.float32)
    g1_ref[...] = h * _dinv(d_ref)


def _tc2_body(sp_ref, g1_ref, d_ref, b1_ref, w2_ref, g2_ref):
    dinv = _dinv(d_ref)
    ssum = sp_ref[0, :N, :] + sp_ref[1, :N, :]
    x2 = dinv * (ssum + g1_ref[...]) + b1_ref[...]
    x2 = jnp.maximum(x2, 0.0)
    h2 = jnp.dot(x2, w2_ref[...], preferred_element_type=jnp.float32)
    g2_ref[...] = h2 * dinv


def _tc3_body(sp_ref, g2_ref, d_ref, b2_ref, out_ref):
    dinv = _dinv(d_ref)
    ssum = sp_ref[0, :N, :] + sp_ref[1, :N, :]
    out_ref[...] = dinv * (ssum + g2_ref[...]) + b2_ref[...]


_f32 = jnp.float32
_tc1 = pl.pallas_call(_tc1_body, out_shape=jax.ShapeDtypeStruct((N, C), _f32))
_tc2 = pl.pallas_call(_tc2_body, out_shape=jax.ShapeDtypeStruct((N, C), _f32))
_tc3 = pl.pallas_call(_tc3_body, out_shape=jax.ShapeDtypeStruct((N, C), _f32))


def kernel(x, edge_index, W1, b1, W2, b2):
    ei = edge_index.astype(jnp.int32)
    src2 = ei[0].reshape(NW, EPW)
    dst2 = ei[1].reshape(NW, EPW)
    srcp = jnp.concatenate(
        [src2, jnp.zeros((NW, EPAD), jnp.int32)], axis=1).reshape(NW, NB, BK)
    dstp = jnp.concatenate(
        [dst2, jnp.full((NW, EPAD), NP - 1, jnp.int32)], axis=1).reshape(NW, NB, BK)
    idx4 = jnp.stack([srcp.reshape(NW, NQ, CH, BK),
                      dstp.reshape(NW, NQ, CH, BK)], axis=2)
    onesC = jnp.ones((BK, C), _f32)
    zerosC = jnp.zeros((NP, C), _f32)
    b1r = b1.reshape(1, C)
    b2r = b2.reshape(1, C)

    deg_kernel = _build_deg_kernel()
    scatter_kernel = _build_scatter_kernel()

    degp = deg_kernel(dstp, onesC, zerosC)
    d2 = jnp.stack([degp[0, :N, 0], degp[1, :N, 0]], axis=1)

    g1 = _tc1(x, W1, d2)
    s1 = scatter_kernel(g1, idx4, zerosC)
    g2 = _tc2(s1, g1, d2, b1r, W2)
    s2 = scatter_kernel(g2, idx4, zerosC)
    out = _tc3(s2, g2, d2, b2r)
    return out
